# Initial kernel scaffold; baseline (speedup 1.0000x reference)
import functools

import jax
import jax.numpy as jnp
from jax import lax
from jax.experimental import pallas as pl
from jax.experimental.pallas import tpu as pltpu
from jax.experimental.pallas import tpu_sc as plsc

N_NODES = 10000
DEG = 32
D = 128
NR = 5

NC = 2
NS = 16
NW = NC * NS
BPW = 320
B_PAD = NW * BPW
G16 = BPW // 16


def _sc_gather(nodes_p, adj, ratings, feature_table):
  mesh = plsc.VectorSubcoreMesh(core_axis_name="c", subcore_axis_name="s")

  @functools.partial(
      pl.kernel,
      mesh=mesh,
      out_type=(
          jax.ShapeDtypeStruct((B_PAD, D), jnp.float32),
          jax.ShapeDtypeStruct((B_PAD, D), jnp.float32),
          jax.ShapeDtypeStruct((B_PAD, DEG), jnp.int32),
      ),
      scratch_types=[
          pltpu.VMEM((BPW,), jnp.int32),
          pltpu.VMEM((BPW, DEG), jnp.int32),
          pltpu.VMEM((DEG, BPW), jnp.int32),
          pltpu.VMEM((BPW, DEG), jnp.int32),
          pltpu.VMEM((BPW, D), jnp.float32),
          pltpu.VMEM((BPW, D), jnp.float32),
          pltpu.SemaphoreType.DMA,
          pltpu.SemaphoreType.DMA,
          pltpu.SemaphoreType.DMA,
          pltpu.SemaphoreType.DMA,
      ],
  )
  def k(nodes_hbm, adj_hbm, rat_hbm, feat_hbm,
        nsum_hbm, self_hbm, rt_hbm,
        idx_v, nb_v, nbt_v, rt_v, self_v, acc_v,
        sem_nb, sem_rt, sem_self, sem_acc):
    wid = lax.axis_index("s") * NC + lax.axis_index("c")
    base = wid * BPW

    pltpu.sync_copy(nodes_hbm.at[pl.ds(base, BPW)], idx_v)
    nb_cp = pltpu.async_copy(adj_hbm.at[idx_v], nb_v, sem_nb)
    rt_cp = pltpu.async_copy(rat_hbm.at[idx_v], rt_v, sem_rt)
    self_cp = pltpu.async_copy(feat_hbm.at[idx_v], self_v, sem_self)

    zero16 = jnp.zeros((16,), jnp.float32)

    def zrow(i, c):
      for k8 in range(D // 16):
        acc_v[i, pl.ds(k8 * 16, 16)] = zero16
      return c

    lax.fori_loop(0, BPW, zrow, 0)

    nb_cp.wait()

    lanes = lax.iota(jnp.int32, 16)

    def tr(j, c):
      col = jnp.full((16,), j, jnp.int32)
      for g in range(G16):
        vals = plsc.load_gather(nb_v, [g * 16 + lanes, col])
        nbt_v[j, pl.ds(g * 16, 16)] = vals
      return c

    lax.fori_loop(0, DEG, tr, 0)

    def fire(j, c):
      pltpu.async_copy(feat_hbm.at[nbt_v.at[j]], acc_v, sem_acc, add=True)
      return c

    lax.fori_loop(0, DEG, fire, 0)

    def drain(j, c):
      pltpu.make_async_copy(feat_hbm.at[nbt_v.at[0]], acc_v, sem_acc).wait()
      return c

    lax.fori_loop(0, DEG, drain, 0)

    rt_cp.wait()
    self_cp.wait()
    pltpu.sync_copy(rt_v, rt_hbm.at[pl.ds(base, BPW)])
    pltpu.sync_copy(self_v, self_hbm.at[pl.ds(base, BPW)])
    pltpu.sync_copy(acc_v, nsum_hbm.at[pl.ds(base, BPW)])

  return k(nodes_p, adj, ratings, feature_table)


BB = 1280


def _tc_body(self_ref, nsum_ref, rt_ref, rtab_ref, wt_ref, b_ref, out_ref):
  rt = rt_ref[...]
  rsum = jnp.zeros((BB, D), jnp.float32)
  for r in range(NR):
    cnt = jnp.sum((rt == r).astype(jnp.float32), axis=1, keepdims=True)
    rsum = rsum + cnt * rtab_ref[r:r + 1, :]
  neigh = (nsum_ref[...] + rsum) * (1.0 / DEG)
  out = jnp.dot(self_ref[...], wt_ref[0:D, :],
                preferred_element_type=jnp.float32)
  out += jnp.dot(neigh, wt_ref[D:2 * D, :],
                 preferred_element_type=jnp.float32)
  out_ref[...] = jnp.maximum(out + b_ref[...], 0.0)


def _tc_combine(selfv, nsum, rtv, rating_table, Wt, b2):
  return pl.pallas_call(
      _tc_body,
      grid=(B_PAD // BB,),
      in_specs=[
          pl.BlockSpec((BB, D), lambda i: (i, 0)),
          pl.BlockSpec((BB, D), lambda i: (i, 0)),
          pl.BlockSpec((BB, DEG), lambda i: (i, 0)),
          pl.BlockSpec((NR, D), lambda i: (0, 0)),
          pl.BlockSpec((2 * D, D), lambda i: (0, 0)),
          pl.BlockSpec((1, D), lambda i: (0, 0)),
      ],
      out_specs=pl.BlockSpec((BB, D), lambda i: (i, 0)),
      out_shape=jax.ShapeDtypeStruct((B_PAD, D), jnp.float32),
  )(selfv, nsum, rtv, rating_table, Wt, b2)


def kernel(nodes, adj, ratings, feature_table, rating_table, W, b):
  nodes = nodes.astype(jnp.int32)
  adj = adj.astype(jnp.int32)
  ratings = ratings.astype(jnp.int32)
  nodes_p = jnp.concatenate(
      [nodes, jnp.zeros((B_PAD - N_NODES,), jnp.int32)])
  nsum, selfv, rtv = _sc_gather(nodes_p, adj, ratings, feature_table)
  out = _tc_combine(selfv, nsum, rtv, rating_table,
                    W.T.astype(jnp.float32), b.reshape(1, D))
  return out[:N_NODES]


# speedup vs baseline: 4.1133x; 4.1133x over previous
import functools

import jax
import jax.numpy as jnp
from jax import lax
from jax.experimental import pallas as pl
from jax.experimental.pallas import tpu as pltpu
from jax.experimental.pallas import tpu_sc as plsc

N_NODES = 10000
DEG = 32
D = 128
NR = 5

NC = 2
NS = 16
NW = NC * NS
BPW = 320
B_PAD = NW * BPW
G16 = BPW // 16
CHUNKS = ((0, 128), (128, 128), (256, 64))


def _sc_gather(nodes_p, ar_pack, feature_table):
  mesh = plsc.VectorSubcoreMesh(core_axis_name="c", subcore_axis_name="s")

  @functools.partial(
      pl.kernel,
      mesh=mesh,
      compiler_params=pltpu.CompilerParams(needs_layout_passes=False),
      out_type=(
          jax.ShapeDtypeStruct((B_PAD, D), jnp.float32),
          jax.ShapeDtypeStruct((B_PAD, D), jnp.float32),
          jax.ShapeDtypeStruct((B_PAD, D), jnp.int32),
      ),
      scratch_types=[
          pltpu.VMEM((BPW,), jnp.int32),
          pltpu.VMEM((BPW, D), jnp.int32),
          pltpu.VMEM((DEG, BPW), jnp.int32),
          pltpu.VMEM((128, D), jnp.float32),
          pltpu.VMEM((BPW, D), jnp.float32),
          pltpu.SemaphoreType.DMA,
          pltpu.SemaphoreType.DMA,
          pltpu.SemaphoreType.DMA,
      ],
  )
  def k(nodes_hbm, ar_hbm, feat_hbm,
        nsum_hbm, self_hbm, aro_hbm,
        idx_v, ar_v, nbt_v, sf_v, acc_v,
        sem_ar, sem_self, sem_acc):
    wid = lax.axis_index("s") * NC + lax.axis_index("c")
    base = wid * BPW

    pltpu.sync_copy(nodes_hbm.at[pl.ds(base, BPW)], idx_v)
    ar_cps = [
        pltpu.async_copy(
            ar_hbm.at[idx_v.at[pl.ds(o, n)]], ar_v.at[pl.ds(o, n)], sem_ar)
        for o, n in CHUNKS
    ]

    zero16 = jnp.zeros((16,), jnp.float32)

    def zrow(i, c):
      for k8 in range(D // 16):
        acc_v[i, pl.ds(k8 * 16, 16)] = zero16
      return c

    lax.fori_loop(0, BPW, zrow, 0)

    for cp in ar_cps:
      cp.wait()

    lanes = lax.iota(jnp.int32, 16)

    def tr(j, c):
      col = jnp.full((16,), j, jnp.int32)
      for g in range(G16):
        vals = plsc.load_gather(ar_v, [g * 16 + lanes, col])
        nbt_v[j, pl.ds(g * 16, 16)] = vals
      return c

    lax.fori_loop(0, DEG, tr, 0)

    def fire(j, c):
      for o, n in CHUNKS:
        pltpu.async_copy(
            feat_hbm.at[nbt_v.at[j, pl.ds(o, n)]],
            acc_v.at[pl.ds(o, n)], sem_acc, add=True)
      return c

    lax.fori_loop(0, DEG, fire, 0)

    pltpu.sync_copy(ar_v, aro_hbm.at[pl.ds(base, BPW)])

    for o, n in CHUNKS:
      pltpu.async_copy(
          feat_hbm.at[idx_v.at[pl.ds(o, n)]], sf_v.at[pl.ds(0, n)],
          sem_self).wait()
      pltpu.sync_copy(sf_v.at[pl.ds(0, n)], self_hbm.at[pl.ds(base + o, n)])

    def drain(j, c):
      for o, n in CHUNKS:
        pltpu.make_async_copy(
            feat_hbm.at[nbt_v.at[0, pl.ds(o, n)]],
            acc_v.at[pl.ds(o, n)], sem_acc).wait()
      return c

    lax.fori_loop(0, DEG, drain, 0)
    pltpu.sync_copy(acc_v, nsum_hbm.at[pl.ds(base, BPW)])

  return k(nodes_p, ar_pack, feature_table)


BB = 1280


def _tc_body(self_ref, nsum_ref, aro_ref, rtab_ref, wt_ref, b_ref, out_ref):
  rt = aro_ref[:, DEG:2 * DEG]
  rsum = jnp.zeros((BB, D), jnp.float32)
  for r in range(NR):
    cnt = jnp.sum((rt == r).astype(jnp.float32), axis=1, keepdims=True)
    rsum = rsum + cnt * rtab_ref[r:r + 1, :]
  neigh = (nsum_ref[...] + rsum) * (1.0 / DEG)
  out = jnp.dot(self_ref[...], wt_ref[0:D, :],
                preferred_element_type=jnp.float32)
  out += jnp.dot(neigh, wt_ref[D:2 * D, :],
                 preferred_element_type=jnp.float32)
  out_ref[...] = jnp.maximum(out + b_ref[...], 0.0)


def _tc_combine(selfv, nsum, aro, rating_table, Wt, b2):
  return pl.pallas_call(
      _tc_body,
      grid=(B_PAD // BB,),
      in_specs=[
          pl.BlockSpec((BB, D), lambda i: (i, 0)),
          pl.BlockSpec((BB, D), lambda i: (i, 0)),
          pl.BlockSpec((BB, D), lambda i: (i, 0)),
          pl.BlockSpec((NR, D), lambda i: (0, 0)),
          pl.BlockSpec((2 * D, D), lambda i: (0, 0)),
          pl.BlockSpec((1, D), lambda i: (0, 0)),
      ],
      out_specs=pl.BlockSpec((BB, D), lambda i: (i, 0)),
      out_shape=jax.ShapeDtypeStruct((B_PAD, D), jnp.float32),
  )(selfv, nsum, aro, rating_table, Wt, b2)


def kernel(nodes, adj, ratings, feature_table, rating_table, W, b):
  nodes = nodes.astype(jnp.int32)
  nodes_p = jnp.concatenate(
      [nodes, jnp.zeros((B_PAD - N_NODES,), jnp.int32)])
  ar_pack = jnp.concatenate(
      [adj.astype(jnp.int32), ratings.astype(jnp.int32),
       jnp.zeros((N_NODES, D - 2 * DEG), jnp.int32)], axis=1)
  nsum, selfv, aro = _sc_gather(nodes_p, ar_pack, feature_table)
  out = _tc_combine(selfv, nsum, aro, rating_table,
                    W.T.astype(jnp.float32), b.reshape(1, D))
  return out[:N_NODES]


# speedup vs baseline: 11.6903x; 2.8421x over previous
import functools

import jax
import jax.numpy as jnp
from jax import lax
from jax.experimental import pallas as pl
from jax.experimental.pallas import tpu as pltpu
from jax.experimental.pallas import tpu_sc as plsc

N_NODES = 10000
DEG = 32
D = 128
NR = 5

NC = 2
NS = 16
NW = NC * NS
BPW = 320
B_PAD = NW * BPW
RPW = 160
G16 = RPW // 16
CHUNKS = ((0, 128), (128, 32))
STAGE_ROWS = 624


def _sc_gather(nodes_p, ar_pack, feature_table):
  mesh = plsc.VectorSubcoreMesh(core_axis_name="c", subcore_axis_name="s")

  @functools.partial(
      pl.kernel,
      mesh=mesh,
      compiler_params=pltpu.CompilerParams(needs_layout_passes=False),
      out_type=(
          jax.ShapeDtypeStruct((B_PAD, D), jnp.float32),
          jax.ShapeDtypeStruct((B_PAD, D), jnp.float32),
          jax.ShapeDtypeStruct((B_PAD, D), jnp.int32),
      ),
      scratch_types=[
          pltpu.VMEM((BPW,), jnp.int32),
          pltpu.VMEM((RPW, D), jnp.int32),
          pltpu.VMEM((DEG, RPW), jnp.int32),
          pltpu.VMEM((RPW, D), jnp.float32),
          pltpu.VMEM_SHARED((N_NODES, D), jnp.float32),
          pltpu.SemaphoreType.DMA,
          pltpu.SemaphoreType.DMA,
          pltpu.SemaphoreType.DMA,
          pltpu.SemaphoreType.DMA,
      ],
  )
  def k(nodes_hbm, ar_hbm, feat_hbm,
        nsum_hbm, self_hbm, aro_hbm,
        idx_v, ar_v, nbt_v, acc_v, feat_s,
        sem_ar, sem_self, sem_acc, sem_stage):
    wid = lax.axis_index("s") * NC + lax.axis_index("c")
    base = wid * BPW
    sid = lax.axis_index("s")

    stage_cp = pltpu.async_copy(
        feat_hbm.at[pl.ds(sid * STAGE_ROWS, STAGE_ROWS)],
        feat_s.at[pl.ds(sid * STAGE_ROWS, STAGE_ROWS)], sem_stage)

    @pl.when(sid == NS - 1)
    def _stage_tail():
      pltpu.sync_copy(
          feat_hbm.at[pl.ds(NS * STAGE_ROWS, N_NODES - NS * STAGE_ROWS)],
          feat_s.at[pl.ds(NS * STAGE_ROWS, N_NODES - NS * STAGE_ROWS)])

    pltpu.sync_copy(nodes_hbm.at[pl.ds(base, BPW)], idx_v)

    zero16 = jnp.zeros((16,), jnp.float32)
    lanes = lax.iota(jnp.int32, 16)

    ar_cps = [
        pltpu.async_copy(
            ar_hbm.at[idx_v.at[pl.ds(o, n)]], ar_v.at[pl.ds(o, n)], sem_ar)
        for o, n in CHUNKS
    ]
    stage_cp.wait()
    plsc.subcore_barrier()

    for r in range(BPW // RPW):
      o0 = r * RPW
      for cp in ar_cps:
        cp.wait()

      def tr(j, c):
        col = jnp.full((16,), j, jnp.int32)
        for g in range(G16):
          vals = plsc.load_gather(ar_v, [g * 16 + lanes, col])
          nbt_v[j, pl.ds(g * 16, 16)] = vals
        return c

      lax.fori_loop(0, DEG, tr, 0)

      pltpu.sync_copy(ar_v, aro_hbm.at[pl.ds(base + o0, RPW)])
      if r + 1 < BPW // RPW:
        o1 = (r + 1) * RPW
        ar_cps = [
            pltpu.async_copy(
                ar_hbm.at[idx_v.at[pl.ds(o1 + o, n)]],
                ar_v.at[pl.ds(o, n)], sem_ar)
            for o, n in CHUNKS
        ]

      for o, n in CHUNKS:
        pltpu.async_copy(
            feat_s.at[idx_v.at[pl.ds(o0 + o, n)]], acc_v.at[pl.ds(o, n)],
            sem_self).wait()
      pltpu.sync_copy(acc_v, self_hbm.at[pl.ds(base + o0, RPW)])

      def zrow(i, c):
        for k8 in range(D // 16):
          acc_v[i, pl.ds(k8 * 16, 16)] = zero16
        return c

      lax.fori_loop(0, RPW, zrow, 0)

      def fire(j, c):
        for o, n in CHUNKS:
          pltpu.async_copy(
              feat_s.at[nbt_v.at[j, pl.ds(o, n)]],
              acc_v.at[pl.ds(o, n)], sem_acc, add=True)
        return c

      lax.fori_loop(0, DEG, fire, 0)

      def drain(j, c):
        for o, n in CHUNKS:
          pltpu.make_async_copy(
              feat_s.at[nbt_v.at[0, pl.ds(o, n)]],
              acc_v.at[pl.ds(o, n)], sem_acc).wait()
        return c

      lax.fori_loop(0, DEG, drain, 0)
      pltpu.sync_copy(acc_v, nsum_hbm.at[pl.ds(base + o0, RPW)])

  return k(nodes_p, ar_pack, feature_table)


BB = 1280


def _tc_body(self_ref, nsum_ref, aro_ref, rtab_ref, wt_ref, b_ref, out_ref):
  rt = aro_ref[:, DEG:2 * DEG]
  rsum = jnp.zeros((BB, D), jnp.float32)
  for r in range(NR):
    cnt = jnp.sum((rt == r).astype(jnp.float32), axis=1, keepdims=True)
    rsum = rsum + cnt * rtab_ref[r:r + 1, :]
  neigh = (nsum_ref[...] + rsum) * (1.0 / DEG)
  out = jnp.dot(self_ref[...], wt_ref[0:D, :],
                preferred_element_type=jnp.float32)
  out += jnp.dot(neigh, wt_ref[D:2 * D, :],
                 preferred_element_type=jnp.float32)
  out_ref[...] = jnp.maximum(out + b_ref[...], 0.0)


def _tc_combine(selfv, nsum, aro, rating_table, Wt, b2):
  return pl.pallas_call(
      _tc_body,
      grid=(B_PAD // BB,),
      in_specs=[
          pl.BlockSpec((BB, D), lambda i: (i, 0)),
          pl.BlockSpec((BB, D), lambda i: (i, 0)),
          pl.BlockSpec((BB, D), lambda i: (i, 0)),
          pl.BlockSpec((NR, D), lambda i: (0, 0)),
          pl.BlockSpec((2 * D, D), lambda i: (0, 0)),
          pl.BlockSpec((1, D), lambda i: (0, 0)),
      ],
      out_specs=pl.BlockSpec((BB, D), lambda i: (i, 0)),
      out_shape=jax.ShapeDtypeStruct((B_PAD, D), jnp.float32),
  )(selfv, nsum, aro, rating_table, Wt, b2)


def kernel(nodes, adj, ratings, feature_table, rating_table, W, b):
  nodes = nodes.astype(jnp.int32)
  nodes_p = jnp.concatenate(
      [nodes, jnp.zeros((B_PAD - N_NODES,), jnp.int32)])
  ar_pack = jnp.concatenate(
      [adj.astype(jnp.int32), ratings.astype(jnp.int32),
       jnp.zeros((N_NODES, D - 2 * DEG), jnp.int32)], axis=1)
  nsum, selfv, aro = _sc_gather(nodes_p, ar_pack, feature_table)
  out = _tc_combine(selfv, nsum, aro, rating_table,
                    W.T.astype(jnp.float32), b.reshape(1, D))
  return out[:N_NODES]


# speedup vs baseline: 12.4690x; 1.0666x over previous
import functools

import jax
import jax.numpy as jnp
from jax import lax
from jax.experimental import pallas as pl
from jax.experimental.pallas import tpu as pltpu
from jax.experimental.pallas import tpu_sc as plsc

N_NODES = 10000
DEG = 32
D = 128
NR = 5

NC = 2
NS = 16
NW = NC * NS
BPW = 320
B_PAD = NW * BPW
RPW = 160
G16 = RPW // 16
CHUNKS = ((0, 128), (128, 32))
STAGE_ROWS = 624


def _sc_gather(nodes_p, ar_pack, feature_table):
  mesh = plsc.VectorSubcoreMesh(core_axis_name="c", subcore_axis_name="s")

  @functools.partial(
      pl.kernel,
      mesh=mesh,
      compiler_params=pltpu.CompilerParams(needs_layout_passes=False),
      out_type=(
          jax.ShapeDtypeStruct((B_PAD, D), jnp.float32),
          jax.ShapeDtypeStruct((B_PAD, D), jnp.float32),
          jax.ShapeDtypeStruct((B_PAD * 8,), jnp.float32),
      ),
      scratch_types=[
          pltpu.VMEM((BPW,), jnp.int32),
          pltpu.VMEM((RPW, D), jnp.int32),
          pltpu.VMEM((DEG, RPW), jnp.int32),
          pltpu.VMEM((RPW, D), jnp.float32),
          pltpu.VMEM((RPW * 8,), jnp.float32),
          pltpu.VMEM_SHARED((N_NODES, D), jnp.float32),
          pltpu.SemaphoreType.DMA,
          pltpu.SemaphoreType.DMA,
          pltpu.SemaphoreType.DMA,
          pltpu.SemaphoreType.DMA,
      ],
  )
  def k(nodes_hbm, ar_hbm, feat_hbm,
        nsum_hbm, self_hbm, aro_hbm,
        idx_v, ar_v, nbt_v, acc_v, cnt_v, feat_s,
        sem_ar, sem_self, sem_acc, sem_stage):
    wid = lax.axis_index("s") * NC + lax.axis_index("c")
    base = wid * BPW
    sid = lax.axis_index("s")

    stage_cp = pltpu.async_copy(
        feat_hbm.at[pl.ds(sid * STAGE_ROWS, STAGE_ROWS)],
        feat_s.at[pl.ds(sid * STAGE_ROWS, STAGE_ROWS)], sem_stage)

    @pl.when(sid == NS - 1)
    def _stage_tail():
      pltpu.sync_copy(
          feat_hbm.at[pl.ds(NS * STAGE_ROWS, N_NODES - NS * STAGE_ROWS)],
          feat_s.at[pl.ds(NS * STAGE_ROWS, N_NODES - NS * STAGE_ROWS)])

    pltpu.sync_copy(nodes_hbm.at[pl.ds(base, BPW)], idx_v)

    zero16 = jnp.zeros((16,), jnp.float32)
    lanes = lax.iota(jnp.int32, 16)

    ar_cps = [
        pltpu.async_copy(
            ar_hbm.at[idx_v.at[pl.ds(o, n)]], ar_v.at[pl.ds(o, n)], sem_ar)
        for o, n in CHUNKS
    ]
    stage_cp.wait()
    plsc.subcore_barrier()

    for r in range(BPW // RPW):
      o0 = r * RPW
      for cp in ar_cps:
        cp.wait()

      def tr(j, c):
        col = jnp.full((16,), j, jnp.int32)
        for g in range(G16):
          vals = plsc.load_gather(ar_v, [g * 16 + lanes, col])
          nbt_v[j, pl.ds(g * 16, 16)] = vals
        return c

      lax.fori_loop(0, DEG, tr, 0)

      for o, n in CHUNKS:
        pltpu.async_copy(
            feat_s.at[idx_v.at[pl.ds(o0 + o, n)]], acc_v.at[pl.ds(o, n)],
            sem_self).wait()
      pltpu.sync_copy(acc_v, self_hbm.at[pl.ds(base + o0, RPW)])

      def zrow(i, c):
        for k8 in range(D // 16):
          acc_v[i, pl.ds(k8 * 16, 16)] = zero16
        return c

      lax.fori_loop(0, RPW, zrow, 0)

      def fire(j, c):
        for o, n in CHUNKS:
          pltpu.async_copy(
              feat_s.at[nbt_v.at[j, pl.ds(o, n)]],
              acc_v.at[pl.ds(o, n)], sem_acc, add=True)
        return c

      lax.fori_loop(0, DEG, fire, 0)

      for g in range(G16):
        rows_g = g * 16 + lanes

        def cbody(j, cs):
          vals = plsc.load_gather(ar_v, [rows_g, jnp.full((16,), DEG, jnp.int32) + j])
          return tuple(
              cs[rr] + (vals == rr).astype(jnp.float32) for rr in range(NR))

        counts = lax.fori_loop(
            0, DEG, cbody, tuple(jnp.zeros((16,), jnp.float32)
                                 for _ in range(NR)))
        for rr in range(NR):
          plsc.store_scatter(cnt_v, [rows_g * 8 + rr], counts[rr])

      pltpu.sync_copy(cnt_v, aro_hbm.at[pl.ds((base + o0) * 8, RPW * 8)])

      if r + 1 < BPW // RPW:
        o1 = (r + 1) * RPW
        ar_cps = [
            pltpu.async_copy(
                ar_hbm.at[idx_v.at[pl.ds(o1 + o, n)]],
                ar_v.at[pl.ds(o, n)], sem_ar)
            for o, n in CHUNKS
        ]

      def drain(j, c):
        for o, n in CHUNKS:
          pltpu.make_async_copy(
              feat_s.at[nbt_v.at[0, pl.ds(o, n)]],
              acc_v.at[pl.ds(o, n)], sem_acc).wait()
        return c

      lax.fori_loop(0, DEG, drain, 0)
      pltpu.sync_copy(acc_v, nsum_hbm.at[pl.ds(base + o0, RPW)])

  return k(nodes_p, ar_pack, feature_table)


BB = 2000


def _tc_body(self_ref, nsum_ref, cnt_ref, rtab_ref, wt_ref, b_ref, out_ref):
  rsum = jnp.zeros((BB, D), jnp.float32)
  for r in range(NR):
    rsum = rsum + cnt_ref[:, r:r + 1] * rtab_ref[r:r + 1, :]
  neigh = (nsum_ref[...] + rsum) * (1.0 / DEG)
  out = jnp.dot(self_ref[...], wt_ref[0:D, :],
                preferred_element_type=jnp.float32)
  out += jnp.dot(neigh, wt_ref[D:2 * D, :],
                 preferred_element_type=jnp.float32)
  out_ref[...] = jnp.maximum(out + b_ref[...], 0.0)


def _tc_combine(selfv, nsum, aro, rating_table, Wt, b2):
  return pl.pallas_call(
      _tc_body,
      grid=(N_NODES // BB,),
      in_specs=[
          pl.BlockSpec((BB, D), lambda i: (i, 0)),
          pl.BlockSpec((BB, D), lambda i: (i, 0)),
          pl.BlockSpec((BB, 8), lambda i: (i, 0)),
          pl.BlockSpec((NR, D), lambda i: (0, 0)),
          pl.BlockSpec((2 * D, D), lambda i: (0, 0)),
          pl.BlockSpec((1, D), lambda i: (0, 0)),
      ],
      out_specs=pl.BlockSpec((BB, D), lambda i: (i, 0)),
      out_shape=jax.ShapeDtypeStruct((N_NODES, D), jnp.float32),
  )(selfv, nsum, aro, rating_table, Wt, b2)


def kernel(nodes, adj, ratings, feature_table, rating_table, W, b):
  nodes = nodes.astype(jnp.int32)
  nodes_p = jnp.concatenate(
      [nodes, jnp.zeros((B_PAD - N_NODES,), jnp.int32)])
  ar_pack = jnp.concatenate(
      [adj.astype(jnp.int32), ratings.astype(jnp.int32),
       jnp.zeros((N_NODES, D - 2 * DEG), jnp.int32)], axis=1)
  nsum, selfv, cnt = _sc_gather(nodes_p, ar_pack, feature_table)
  return _tc_combine(selfv, nsum, cnt.reshape(B_PAD, 8), rating_table,
                     W.T.astype(jnp.float32), b.reshape(1, D))
